# baseline (speedup 1.0000x reference)
import functools

import jax
import jax.numpy as jnp
from jax import lax
from jax.experimental import pallas as pl
from jax.experimental.pallas import tpu as pltpu
from jax.experimental.pallas import tpu_sc as plsc

N = 10000
D_IN = 128
D_HID = 16
D_OUT = 128

NC = 2
NS = 16
NW = NC * NS
CHUNK = 128
NCH = 2500
CPT = NCH // NW
CREM = NCH % NW
UPT = N // NS

_MESH = dict(core_axis_name="c", subcore_axis_name="s")
_SC_PARAMS = pltpu.CompilerParams(use_tc_tiling_on_sc=False,
                                  needs_layout_passes=False)


def _tile_range(wid):
    cnt = jnp.where(wid < CREM, CPT + 1, CPT)
    base = wid * CPT + jnp.minimum(wid, CREM)
    return base, cnt


def _load_chunks(ei_hbm, which, wid, idx_v):
    base, _ = _tile_range(wid)

    @pl.when(wid < CREM)
    def _():
        pltpu.sync_copy(ei_hbm.at[which, pl.ds(base, CPT + 1)], idx_v)

    @pl.when(wid >= CREM)
    def _():
        pltpu.sync_copy(ei_hbm.at[which, pl.ds(base, CPT)],
                        idx_v.at[pl.ds(0, CPT)])


DEG_ROWS = 10240
DPT = DEG_ROWS // NS


GRP = 8


def _edge_loop(ei_hbm, wid, srcv, dstv, rows_v, u_sh, acc_sh, sem_g, sem_s):
    _load_chunks(ei_hbm, 0, wid, srcv)
    _load_chunks(ei_hbm, 1, wid, dstv)
    plsc.subcore_barrier()
    _, cnt = _tile_range(wid)
    n_grp = cnt // GRP

    for k in range(GRP):
        pltpu.async_copy(u_sh.at[srcv.at[k]], rows_v.at[0, k], sem_g)

    def body(g, carry):
        j0 = g * GRP
        p = lax.rem(g, 2)
        for k in range(GRP):
            pltpu.make_async_copy(
                u_sh.at[srcv.at[0]], rows_v.at[p, k], sem_g).wait()
        for k in range(GRP):
            pltpu.async_copy(
                rows_v.at[p, k], acc_sh.at[dstv.at[j0 + k]], sem_s,
                add=True)

        @pl.when(g + 1 < n_grp)
        def _():
            for k in range(GRP):
                pltpu.async_copy(
                    u_sh.at[srcv.at[j0 + GRP + k]], rows_v.at[1 - p, k],
                    sem_g)

        for k in range(GRP):
            pltpu.make_async_copy(
                rows_v.at[p, k], acc_sh.at[dstv.at[j0]], sem_s).wait()
        return carry

    lax.fori_loop(0, n_grp, body, 0)

    def tail(j, carry):
        pltpu.async_copy(u_sh.at[srcv.at[j]], rows_v.at[0, 0], sem_g).wait()
        pltpu.sync_copy(rows_v.at[0, 0], acc_sh.at[dstv.at[j]], add=True)
        return carry

    lax.fori_loop(n_grp * GRP, cnt, tail, 0)
    plsc.subcore_barrier()


DGB = 648


def _head_dinv(deg_hbm, dgv, dinv_v, s):
    start = s * UPT
    astart = (start // 8) * 8
    off = start - astart
    pltpu.sync_copy(deg_hbm.at[0, pl.ds(astart, DGB)], dgv)
    _dinv_from(dgv, off, dinv_v)


def _dinv_from(dgv, off, dinv_v):
    def ch(t, carry):
        d = dgv[pl.ds(off + 16 * t, 16)] + 1.0
        i32 = plsc.bitcast(d, jnp.int32)
        i32 = 0x5F3759DF - lax.shift_right_logical(i32, 1)
        y = plsc.bitcast(i32, jnp.float32)
        y = y * (1.5 - 0.5 * d * y * y)
        y = y * (1.5 - 0.5 * d * y * y)
        y = y * (1.5 - 0.5 * d * y * y)
        dinv_v[pl.ds(16 * t, 16)] = y
        return carry

    lax.fori_loop(0, (DGB - 8) // 16, ch, 0)


def _bcast_row(dinv_v, i):
    return plsc.load_gather(dinv_v, [jnp.full((16,), i, jnp.int32)])

def _zero_acc(stage_v, acc_sh, s):
    def fill_zero(i, carry):
        stage_v[i] = jnp.zeros((D_HID,), jnp.float32)
        return carry

    lax.fori_loop(0, UPT, fill_zero, 0)
    pltpu.sync_copy(stage_v, acc_sh.at[pl.ds(s * UPT, UPT)])


DPC = NCH // NS
DREM = NCH % NS


def _sc_edge1(p1, ei3):
    @functools.partial(
        pl.kernel,
        out_type=(jax.ShapeDtypeStruct((NC, N, D_HID), jnp.float32),
                  jax.ShapeDtypeStruct((NC, DEG_ROWS), jnp.float32)),
        mesh=plsc.VectorSubcoreMesh(**_MESH),
        scratch_types=[
            pltpu.VMEM_SHARED((N, D_HID), jnp.float32),
            pltpu.VMEM_SHARED((N, D_HID), jnp.float32),
            pltpu.VMEM_SHARED((DEG_ROWS,), jnp.float32),
            pltpu.VMEM((CPT + 1, CHUNK), jnp.int32),
            pltpu.VMEM((CPT + 1, CHUNK), jnp.int32),
            pltpu.VMEM((2, GRP, CHUNK, D_HID), jnp.float32),
            pltpu.VMEM((UPT, D_HID), jnp.float32),
            pltpu.VMEM((UPT, D_HID), jnp.float32),
            pltpu.VMEM((DGB,), jnp.float32),
            pltpu.VMEM((DGB - 8,), jnp.float32),
            pltpu.VMEM((CHUNK,), jnp.float32),
            pltpu.SemaphoreType.DMA,
            pltpu.SemaphoreType.DMA,
        ],
        compiler_params=_SC_PARAMS,
    )
    def k(p_hbm, ei_hbm, out_hbm, deg_out_hbm,
          acc_sh, u_sh, deg_sh, srcv, dstv, rows_v, ubuf, dbuf,
          dgv, dinv_v, ones_v, sem_g, sem_s):
        c = lax.axis_index("c")
        s = lax.axis_index("s")
        wid = c * NS + s

        def fill_zero(i, carry):
            dgv[pl.ds(i * 16, 16)] = jnp.zeros((16,), jnp.float32)
            return carry

        lax.fori_loop(0, DPT // 16, fill_zero, 0)

        def fill_one(i, carry):
            ones_v[pl.ds(i * 16, 16)] = jnp.ones((16,), jnp.float32)
            return carry

        lax.fori_loop(0, CHUNK // 16, fill_one, 0)

        pltpu.sync_copy(dgv.at[pl.ds(0, DPT)], deg_sh.at[pl.ds(s * DPT, DPT)])
        pltpu.sync_copy(p_hbm.at[pl.ds(s * UPT, UPT)], ubuf)
        plsc.subcore_barrier()

        dbase = s * DPC + jnp.minimum(s, DREM)
        for blk in range(2):
            pltpu.sync_copy(ei_hbm.at[1, pl.ds(dbase + blk * 78, 78)],
                            dstv.at[pl.ds(0, 78)])

            def dbody(g, carry):
                j0 = g * 8
                for kk in range(8):
                    pltpu.async_copy(ones_v, deg_sh.at[dstv.at[j0 + kk]],
                                     sem_s, add=True)
                for kk in range(8):
                    pltpu.make_async_copy(
                        ones_v, deg_sh.at[dstv.at[j0]], sem_s).wait()
                return carry

            lax.fori_loop(0, 78 // 8, dbody, 0)

            def dtail(j, carry):
                pltpu.sync_copy(ones_v, deg_sh.at[dstv.at[j]], add=True)
                return carry

            lax.fori_loop((78 // 8) * 8, 78, dtail, 0)

        @pl.when(s < DREM)
        def _():
            pltpu.sync_copy(ei_hbm.at[1, pl.ds(dbase + 2 * 78, 1)],
                            dstv.at[pl.ds(0, 1)])
            pltpu.sync_copy(ones_v, deg_sh.at[dstv.at[0]], add=True)

        plsc.subcore_barrier()

        pltpu.sync_copy(deg_sh.at[pl.ds(s * DPT, DPT)], dgv.at[pl.ds(0, DPT)])
        pltpu.sync_copy(dgv.at[pl.ds(0, DPT)],
                        deg_out_hbm.at[c, pl.ds(s * DPT, DPT)])

        start = s * UPT
        astart = (start // 8) * 8
        off = start - astart
        pltpu.sync_copy(deg_sh.at[pl.ds(astart, DGB)], dgv)
        _dinv_from(dgv, off, dinv_v)

        def scale(i, carry):
            ubuf[i] = ubuf[i] * _bcast_row(dinv_v, i)
            return carry

        lax.fori_loop(0, UPT, scale, 0)
        pltpu.sync_copy(ubuf, u_sh.at[pl.ds(s * UPT, UPT)])
        _zero_acc(dbuf, acc_sh, s)

        _edge_loop(ei_hbm, wid, srcv, dstv, rows_v, u_sh, acc_sh, sem_g, sem_s)

        pltpu.sync_copy(acc_sh.at[pl.ds(s * UPT, UPT)], ubuf)
        pltpu.sync_copy(ubuf, out_hbm.at[c, pl.ds(s * UPT, UPT)])

    return k(p1, ei3)


def _sc_edge2(s1, p1, deg_part, b1, ei3):
    @functools.partial(
        pl.kernel,
        out_type=jax.ShapeDtypeStruct((NC, N, D_HID), jnp.float32),
        mesh=plsc.VectorSubcoreMesh(**_MESH),
        scratch_types=[
            pltpu.VMEM_SHARED((N, D_HID), jnp.float32),
            pltpu.VMEM_SHARED((N, D_HID), jnp.float32),
            pltpu.VMEM((CPT + 1, CHUNK), jnp.int32),
            pltpu.VMEM((CPT + 1, CHUNK), jnp.int32),
            pltpu.VMEM((2, GRP, CHUNK, D_HID), jnp.float32),
            pltpu.VMEM((UPT, D_HID), jnp.float32),
            pltpu.VMEM((UPT, D_HID), jnp.float32),
            pltpu.VMEM((UPT, D_HID), jnp.float32),
            pltpu.VMEM((UPT, D_HID), jnp.float32),
            pltpu.VMEM((16,), jnp.float32),
            pltpu.VMEM((DGB,), jnp.float32),
            pltpu.VMEM((DGB - 8,), jnp.float32),
            pltpu.SemaphoreType.DMA,
            pltpu.SemaphoreType.DMA,
        ],
        compiler_params=_SC_PARAMS,
    )
    def k(s1_hbm, p_hbm, deg_hbm, b_hbm, ei_hbm, out_hbm,
          acc_sh, u_sh, srcv, dstv, rows_v, ubuf, dbuf, t0, t1, bv,
          dgv, dinv_v, sem_g, sem_s):
        c = lax.axis_index("c")
        s = lax.axis_index("s")
        wid = c * NS + s

        pltpu.sync_copy(p_hbm.at[pl.ds(s * UPT, UPT)], ubuf)
        pltpu.sync_copy(s1_hbm.at[0, pl.ds(s * UPT, UPT)], t0)
        pltpu.sync_copy(s1_hbm.at[1, pl.ds(s * UPT, UPT)], t1)
        pltpu.sync_copy(b_hbm, bv)
        bias = bv[...]
        _head_dinv(deg_hbm, dgv, dinv_v, s)

        def head(i, carry):
            d = _bcast_row(dinv_v, i)
            agg = d * (t0[i] + t1[i] + ubuf[i] * d)
            ubuf[i] = jnp.maximum(agg + bias, 0.0) * d
            return carry

        lax.fori_loop(0, UPT, head, 0)
        pltpu.sync_copy(ubuf, u_sh.at[pl.ds(s * UPT, UPT)])
        _zero_acc(dbuf, acc_sh, s)

        _edge_loop(ei_hbm, wid, srcv, dstv, rows_v, u_sh, acc_sh, sem_g, sem_s)

        pltpu.sync_copy(acc_sh.at[pl.ds(s * UPT, UPT)], ubuf)
        pltpu.sync_copy(ubuf, out_hbm.at[c, pl.ds(s * UPT, UPT)])

    return k(s1, p1, deg_part, b1, ei3)


def _tc_mm(x, w1):
    def body(x_ref, w_ref, p_ref):
        p_ref[...] = jnp.dot(x_ref[...], w_ref[...],
                             preferred_element_type=jnp.float32)

    return pl.pallas_call(
        body, out_shape=jax.ShapeDtypeStruct((N, D_HID), jnp.float32),
    )(x, w1)


def _tc_out(s1, s2, p1, deg_t, b1, w2, b2):
    def body(s1_ref, s2_ref, p_ref, dt_ref, b1_ref, w_ref, b2_ref, out_ref):
        d = lax.rsqrt(dt_ref[...] + 1.0)
        u1 = p_ref[...] * d
        u2 = jnp.maximum(d * (s1_ref[0] + s1_ref[1] + u1) + b1_ref[...],
                         0.0) * d
        agg = d * (s2_ref[0] + s2_ref[1] + u2)
        out_ref[...] = (
            jnp.dot(agg, w_ref[...], preferred_element_type=jnp.float32)
            + b2_ref[...])

    return pl.pallas_call(
        body, out_shape=jax.ShapeDtypeStruct((N, D_OUT), jnp.float32),
    )(s1, s2, p1, deg_t, b1, w2, b2)


def kernel(x_graph, edge_index, W1, b1, W2, b2):
    ei3 = edge_index.reshape(2, NCH, CHUNK)

    p1 = _tc_mm(x_graph, W1)

    s1, deg_out = _sc_edge1(p1, ei3)
    s2 = _sc_edge2(s1, p1, deg_out, b1, ei3)
    return _tc_out(s1, s2, p1, deg_out[0, :N].reshape(N, 1),
                   b1.reshape(1, D_HID), W2, b2.reshape(1, D_OUT))


# speedup vs baseline: 1.0840x; 1.0840x over previous
import functools

import jax
import jax.numpy as jnp
from jax import lax
from jax.experimental import pallas as pl
from jax.experimental.pallas import tpu as pltpu
from jax.experimental.pallas import tpu_sc as plsc

N = 10000
D_IN = 128
D_HID = 16
D_OUT = 128

NC = 2
NS = 16
NW = NC * NS
CHUNK = 128
NCH = 2500
CPT = NCH // NW
CREM = NCH % NW
UPT = N // NS

_MESH = dict(core_axis_name="c", subcore_axis_name="s")
_SC_PARAMS = pltpu.CompilerParams(use_tc_tiling_on_sc=False,
                                  needs_layout_passes=False)


def _tile_range(wid):
    cnt = jnp.where(wid < CREM, CPT + 1, CPT)
    base = wid * CPT + jnp.minimum(wid, CREM)
    return base, cnt


def _load_chunks(ei_hbm, which, wid, idx_v):
    base, _ = _tile_range(wid)

    @pl.when(wid < CREM)
    def _():
        pltpu.sync_copy(ei_hbm.at[which, pl.ds(base, CPT + 1)], idx_v)

    @pl.when(wid >= CREM)
    def _():
        pltpu.sync_copy(ei_hbm.at[which, pl.ds(base, CPT)],
                        idx_v.at[pl.ds(0, CPT)])


DEG_ROWS = 10240
DPT = DEG_ROWS // NS


def _sc_deg(ei3):
    @functools.partial(
        pl.kernel,
        out_type=jax.ShapeDtypeStruct((NC, DEG_ROWS), jnp.float32),
        mesh=plsc.VectorSubcoreMesh(**_MESH),
        scratch_types=[
            pltpu.VMEM_SHARED((DEG_ROWS,), jnp.float32),
            pltpu.VMEM((CPT + 1, CHUNK), jnp.int32),
            pltpu.VMEM((CHUNK,), jnp.float32),
            pltpu.VMEM((DPT,), jnp.float32),
            pltpu.SemaphoreType.DMA,
        ],
        compiler_params=_SC_PARAMS,
    )
    def k(ei_hbm, out_hbm, acc_sh, idx_v, ones_v, stage_v, sem):
        c = lax.axis_index("c")
        s = lax.axis_index("s")
        wid = c * NS + s
        _, cnt = _tile_range(wid)

        def fill_zero(i, carry):
            stage_v[pl.ds(i * 16, 16)] = jnp.zeros((16,), jnp.float32)
            return carry

        lax.fori_loop(0, DPT // 16, fill_zero, 0)

        def fill_one(i, carry):
            ones_v[pl.ds(i * 16, 16)] = jnp.ones((16,), jnp.float32)
            return carry

        lax.fori_loop(0, CHUNK // 16, fill_one, 0)

        pltpu.sync_copy(stage_v, acc_sh.at[pl.ds(s * DPT, DPT)])
        _load_chunks(ei_hbm, 1, wid, idx_v)
        plsc.subcore_barrier()

        n_grp = cnt // 8

        def body(g, carry):
            j0 = g * 8
            for k in range(8):
                pltpu.async_copy(ones_v, acc_sh.at[idx_v.at[j0 + k]], sem,
                                 add=True)
            for k in range(8):
                pltpu.make_async_copy(
                    ones_v, acc_sh.at[idx_v.at[j0]], sem).wait()
            return carry

        lax.fori_loop(0, n_grp, body, 0)

        def tail(j, carry):
            pltpu.sync_copy(ones_v, acc_sh.at[idx_v.at[j]], add=True)
            return carry

        lax.fori_loop(n_grp * 8, cnt, tail, 0)
        plsc.subcore_barrier()
        pltpu.sync_copy(acc_sh.at[pl.ds(s * DPT, DPT)], stage_v)
        pltpu.sync_copy(stage_v, out_hbm.at[c, pl.ds(s * DPT, DPT)])

    return k(ei3)


GRP = 8


def _edge_loop(ei_hbm, wid, srcv, dstv, rows_v, u_sh, acc_sh, sem_g, sem_s):
    _load_chunks(ei_hbm, 0, wid, srcv)
    _load_chunks(ei_hbm, 1, wid, dstv)
    plsc.subcore_barrier()
    _, cnt = _tile_range(wid)
    n_grp = cnt // GRP

    for k in range(GRP):
        pltpu.async_copy(u_sh.at[srcv.at[k]], rows_v.at[0, k], sem_g)

    def body(g, carry):
        j0 = g * GRP
        p = lax.rem(g, 2)
        for k in range(GRP):
            pltpu.make_async_copy(
                u_sh.at[srcv.at[0]], rows_v.at[p, k], sem_g).wait()
        for k in range(GRP):
            pltpu.async_copy(
                rows_v.at[p, k], acc_sh.at[dstv.at[j0 + k]], sem_s,
                add=True)

        @pl.when(g + 1 < n_grp)
        def _():
            for k in range(GRP):
                pltpu.async_copy(
                    u_sh.at[srcv.at[j0 + GRP + k]], rows_v.at[1 - p, k],
                    sem_g)

        for k in range(GRP):
            pltpu.make_async_copy(
                rows_v.at[p, k], acc_sh.at[dstv.at[j0]], sem_s).wait()
        return carry

    lax.fori_loop(0, n_grp, body, 0)

    def tail(j, carry):
        pltpu.async_copy(u_sh.at[srcv.at[j]], rows_v.at[0, 0], sem_g).wait()
        pltpu.sync_copy(rows_v.at[0, 0], acc_sh.at[dstv.at[j]], add=True)
        return carry

    lax.fori_loop(n_grp * GRP, cnt, tail, 0)
    plsc.subcore_barrier()


DGB = 648


def _head_dinv(deg_hbm, dg0, dg1, dinv_v, s):
    start = s * UPT
    astart = (start // 8) * 8
    off = start - astart
    pltpu.sync_copy(deg_hbm.at[0, pl.ds(astart, DGB)], dg0)
    pltpu.sync_copy(deg_hbm.at[1, pl.ds(astart, DGB)], dg1)

    def ch(t, carry):
        d = dg0[pl.ds(off + 16 * t, 16)] + dg1[pl.ds(off + 16 * t, 16)] + 1.0
        i32 = plsc.bitcast(d, jnp.int32)
        i32 = 0x5F3759DF - lax.shift_right_logical(i32, 1)
        y = plsc.bitcast(i32, jnp.float32)
        y = y * (1.5 - 0.5 * d * y * y)
        y = y * (1.5 - 0.5 * d * y * y)
        y = y * (1.5 - 0.5 * d * y * y)
        dinv_v[pl.ds(16 * t, 16)] = y
        return carry

    lax.fori_loop(0, (DGB - 8) // 16, ch, 0)


def _bcast_row(dinv_v, i):
    return plsc.load_gather(dinv_v, [jnp.full((16,), i, jnp.int32)])

def _zero_acc(stage_v, acc_sh, s):
    def fill_zero(i, carry):
        stage_v[i] = jnp.zeros((D_HID,), jnp.float32)
        return carry

    lax.fori_loop(0, UPT, fill_zero, 0)
    pltpu.sync_copy(stage_v, acc_sh.at[pl.ds(s * UPT, UPT)])


def _sc_edge1(p1, deg_part, ei3):
    @functools.partial(
        pl.kernel,
        out_type=jax.ShapeDtypeStruct((NC, N, D_HID), jnp.float32),
        mesh=plsc.VectorSubcoreMesh(**_MESH),
        scratch_types=[
            pltpu.VMEM_SHARED((N, D_HID), jnp.float32),
            pltpu.VMEM_SHARED((N, D_HID), jnp.float32),
            pltpu.VMEM((CPT + 1, CHUNK), jnp.int32),
            pltpu.VMEM((CPT + 1, CHUNK), jnp.int32),
            pltpu.VMEM((2, GRP, CHUNK, D_HID), jnp.float32),
            pltpu.VMEM((UPT, D_HID), jnp.float32),
            pltpu.VMEM((UPT, D_HID), jnp.float32),
            pltpu.VMEM((DGB,), jnp.float32),
            pltpu.VMEM((DGB,), jnp.float32),
            pltpu.VMEM((DGB - 8,), jnp.float32),
            pltpu.SemaphoreType.DMA,
            pltpu.SemaphoreType.DMA,
        ],
        compiler_params=_SC_PARAMS,
    )
    def k(p_hbm, deg_hbm, ei_hbm, out_hbm,
          acc_sh, u_sh, srcv, dstv, rows_v, ubuf, dbuf,
          dg0, dg1, dinv_v, sem_g, sem_s):
        c = lax.axis_index("c")
        s = lax.axis_index("s")
        wid = c * NS + s

        pltpu.sync_copy(p_hbm.at[pl.ds(s * UPT, UPT)], ubuf)
        _head_dinv(deg_hbm, dg0, dg1, dinv_v, s)

        def scale(i, carry):
            ubuf[i] = ubuf[i] * _bcast_row(dinv_v, i)
            return carry

        lax.fori_loop(0, UPT, scale, 0)
        pltpu.sync_copy(ubuf, u_sh.at[pl.ds(s * UPT, UPT)])
        _zero_acc(dbuf, acc_sh, s)

        _edge_loop(ei_hbm, wid, srcv, dstv, rows_v, u_sh, acc_sh, sem_g, sem_s)

        pltpu.sync_copy(acc_sh.at[pl.ds(s * UPT, UPT)], ubuf)
        pltpu.sync_copy(ubuf, out_hbm.at[c, pl.ds(s * UPT, UPT)])

    return k(p1, deg_part, ei3)


def _sc_edge2(s1, p1, deg_part, b1, ei3):
    @functools.partial(
        pl.kernel,
        out_type=jax.ShapeDtypeStruct((NC, N, D_HID), jnp.float32),
        mesh=plsc.VectorSubcoreMesh(**_MESH),
        scratch_types=[
            pltpu.VMEM_SHARED((N, D_HID), jnp.float32),
            pltpu.VMEM_SHARED((N, D_HID), jnp.float32),
            pltpu.VMEM((CPT + 1, CHUNK), jnp.int32),
            pltpu.VMEM((CPT + 1, CHUNK), jnp.int32),
            pltpu.VMEM((2, GRP, CHUNK, D_HID), jnp.float32),
            pltpu.VMEM((UPT, D_HID), jnp.float32),
            pltpu.VMEM((UPT, D_HID), jnp.float32),
            pltpu.VMEM((UPT, D_HID), jnp.float32),
            pltpu.VMEM((UPT, D_HID), jnp.float32),
            pltpu.VMEM((16,), jnp.float32),
            pltpu.VMEM((DGB,), jnp.float32),
            pltpu.VMEM((DGB,), jnp.float32),
            pltpu.VMEM((DGB - 8,), jnp.float32),
            pltpu.SemaphoreType.DMA,
            pltpu.SemaphoreType.DMA,
        ],
        compiler_params=_SC_PARAMS,
    )
    def k(s1_hbm, p_hbm, deg_hbm, b_hbm, ei_hbm, out_hbm,
          acc_sh, u_sh, srcv, dstv, rows_v, ubuf, dbuf, t0, t1, bv,
          dg0, dg1, dinv_v, sem_g, sem_s):
        c = lax.axis_index("c")
        s = lax.axis_index("s")
        wid = c * NS + s

        pltpu.sync_copy(p_hbm.at[pl.ds(s * UPT, UPT)], ubuf)
        pltpu.sync_copy(s1_hbm.at[0, pl.ds(s * UPT, UPT)], t0)
        pltpu.sync_copy(s1_hbm.at[1, pl.ds(s * UPT, UPT)], t1)
        pltpu.sync_copy(b_hbm, bv)
        bias = bv[...]
        _head_dinv(deg_hbm, dg0, dg1, dinv_v, s)

        def head(i, carry):
            d = _bcast_row(dinv_v, i)
            agg = d * (t0[i] + t1[i] + ubuf[i] * d)
            ubuf[i] = jnp.maximum(agg + bias, 0.0) * d
            return carry

        lax.fori_loop(0, UPT, head, 0)
        pltpu.sync_copy(ubuf, u_sh.at[pl.ds(s * UPT, UPT)])
        _zero_acc(dbuf, acc_sh, s)

        _edge_loop(ei_hbm, wid, srcv, dstv, rows_v, u_sh, acc_sh, sem_g, sem_s)

        pltpu.sync_copy(acc_sh.at[pl.ds(s * UPT, UPT)], ubuf)
        pltpu.sync_copy(ubuf, out_hbm.at[c, pl.ds(s * UPT, UPT)])

    return k(s1, p1, deg_part, b1, ei3)


def _tc_mm(x, w1):
    def body(x_ref, w_ref, p_ref):
        p_ref[...] = jnp.dot(x_ref[...], w_ref[...],
                             preferred_element_type=jnp.float32)

    return pl.pallas_call(
        body, out_shape=jax.ShapeDtypeStruct((N, D_HID), jnp.float32),
    )(x, w1)


def _tc_out(s1, s2, p1, deg_t, b1, w2, b2):
    BLK = 2000

    def body(s1_ref, s2_ref, p_ref, dt_ref, b1_ref, w_ref, b2_ref, out_ref):
        d = lax.rsqrt(dt_ref[:, 0:1] + dt_ref[:, 1:2] + 1.0)
        u1 = p_ref[...] * d
        u2 = jnp.maximum(d * (s1_ref[0] + s1_ref[1] + u1) + b1_ref[...],
                         0.0) * d
        agg = d * (s2_ref[0] + s2_ref[1] + u2)
        out_ref[...] = (
            jnp.dot(agg, w_ref[...], preferred_element_type=jnp.float32)
            + b2_ref[...])

    return pl.pallas_call(
        body,
        grid=(N // BLK,),
        in_specs=[
            pl.BlockSpec((NC, BLK, D_HID), lambda i: (0, i, 0)),
            pl.BlockSpec((NC, BLK, D_HID), lambda i: (0, i, 0)),
            pl.BlockSpec((BLK, D_HID), lambda i: (i, 0)),
            pl.BlockSpec((BLK, NC), lambda i: (i, 0)),
            pl.BlockSpec((1, D_HID), lambda i: (0, 0)),
            pl.BlockSpec((D_HID, D_OUT), lambda i: (0, 0)),
            pl.BlockSpec((1, D_OUT), lambda i: (0, 0)),
        ],
        out_specs=pl.BlockSpec((BLK, D_OUT), lambda i: (i, 0)),
        out_shape=jax.ShapeDtypeStruct((N, D_OUT), jnp.float32),
        compiler_params=pltpu.CompilerParams(
            dimension_semantics=("arbitrary",)),
    )(s1, s2, p1, deg_t, b1, w2, b2)


def kernel(x_graph, edge_index, W1, b1, W2, b2):
    ei3 = edge_index.reshape(2, NCH, CHUNK)

    deg_part = _sc_deg(ei3)
    p1 = _tc_mm(x_graph, W1)

    s1 = _sc_edge1(p1, deg_part, ei3)
    s2 = _sc_edge2(s1, p1, deg_part, b1, ei3)
    return _tc_out(s1, s2, p1, deg_part[:, :N].T,
                   b1.reshape(1, D_HID), W2, b2.reshape(1, D_OUT))
